# baseline (device time: 74944 ns/iter reference)
import jax
import jax.numpy as jnp
from jax import lax
from jax.experimental import pallas as pl
from jax.experimental.pallas import tpu as pltpu

N_DEV = 16
S = 2

RING = [0, 1, 5, 9, 13, 14, 10, 6, 2, 3, 7, 11, 15, 12, 8, 4]

CW_STEPS = N_DEV // 2
CCW_STEPS = N_DEV // 2 - 1


def kernel(A, B):
    m, k = A.shape
    _, n = B.shape
    chunk = m // N_DEV
    S_eff = max(1, min(S, n // 256))
    w = n // S_eff

    my = lax.axis_index("i")
    r_arr = jnp.array(RING, jnp.int32)
    kpos = jnp.argmax(r_arr == my).astype(jnp.int32)
    succ = r_arr[(kpos + 1) % N_DEV]
    pred = r_arr[(kpos - 1) % N_DEV]
    offs = jnp.array([8] + [7 - h for h in range(CW_STEPS)]
                     + [-7] + [-6 + h for h in range(CCW_STEPS)], jnp.int32)
    ids = r_arr[(kpos + offs) % N_DEV]
    params = jnp.concatenate([succ[None], pred[None], ids]).astype(jnp.int32)

    def body(a_ref, b_ref, prm_ref, out_ref, ab_ref, bb_ref, p_ref,
             *stream_refs):
        succ = prm_ref[0]
        pred = prm_ref[1]

        streams = []
        metas = [(d, j2 * w) for j2 in range(S_eff) for d in (0, 1)]
        for j, (d, lo) in enumerate(metas):
            buf, snd, rcv = stream_refs[3 * j: 3 * j + 3]
            streams.append(dict(
                buf=buf, snd=snd, rcv=rcv, d=d, lo=lo,
                dst=succ if d == 0 else pred,
                steps=CW_STEPS if d == 0 else CCW_STEPS,
                acc_base=3 if d == 0 else 12,
                init_idx=2 if d == 0 else 11,
                rdmas=[],
            ))

        barrier_sem = pltpu.get_barrier_semaphore()
        for nbr in (succ, pred):
            pl.semaphore_signal(barrier_sem, inc=1, device_id=(nbr,),
                                device_id_type=pl.DeviceIdType.MESH)
        pl.semaphore_wait(barrier_sem, 2)

        ab_ref[:, :] = a_ref[:, :].astype(jnp.bfloat16)
        bb_ref[:, :] = b_ref[:, :].astype(jnp.bfloat16)
        p_ref[:, :] = jnp.dot(
            ab_ref[:, :], bb_ref[:, :],
            preferred_element_type=jnp.float32,
        ).astype(jnp.bfloat16)

        def own(c, lo):
            return p_ref[pl.ds(c * chunk, chunk), lo:lo + w]

        def start_hop(s, h):
            rd = pltpu.make_async_remote_copy(
                src_ref=s['buf'].at[h],
                dst_ref=s['buf'].at[h + 1],
                send_sem=s['snd'].at[h],
                recv_sem=s['rcv'].at[h],
                device_id=(s['dst'],),
                device_id_type=pl.DeviceIdType.MESH,
            )
            rd.start()
            s['rdmas'].append(rd)

        for s in streams:
            s['buf'][0, :, :] = own(prm_ref[s['init_idx']], s['lo'])
        for s in streams:
            start_hop(s, 0)

        for h in range(CW_STEPS):
            for s in streams:
                if h >= s['steps']:
                    continue
                s['rdmas'][h].wait_recv()
                if h < s['steps'] - 1:
                    s['buf'][h + 1, :, :] = (
                        s['buf'][h + 1, :, :]
                        + own(prm_ref[s['acc_base'] + h], s['lo']))
                    start_hop(s, h + 1)

        for j, (d, lo) in enumerate(metas):
            if d != 0:
                continue
            s_cw = streams[j]
            s_ccw = streams[j + 1]
            assert s_ccw['d'] == 1 and s_ccw['lo'] == lo
            out_ref[:, lo:lo + w] = (
                s_cw['buf'][CW_STEPS, :, :].astype(jnp.float32)
                + s_ccw['buf'][CCW_STEPS, :, :].astype(jnp.float32)
                + own(prm_ref[10], lo).astype(jnp.float32))

        for s in streams:
            for rd in s['rdmas']:
                rd.wait_send()

    stream_scratch = []
    for d_, lo_ in [(d, j2 * w) for j2 in range(S_eff) for d in (0, 1)]:
        steps = CW_STEPS if d_ == 0 else CCW_STEPS
        stream_scratch += [
            pltpu.VMEM((steps + 1, chunk, w), jnp.bfloat16),
            pltpu.SemaphoreType.DMA((steps,)),
            pltpu.SemaphoreType.DMA((steps,)),
        ]

    return pl.pallas_call(
        body,
        out_shape=jax.ShapeDtypeStruct((chunk, n), jnp.float32),
        in_specs=[
            pl.BlockSpec(memory_space=pltpu.VMEM),
            pl.BlockSpec(memory_space=pltpu.VMEM),
            pl.BlockSpec(memory_space=pltpu.SMEM),
        ],
        out_specs=pl.BlockSpec(memory_space=pltpu.VMEM),
        scratch_shapes=[
            pltpu.VMEM((m, k), jnp.bfloat16),
            pltpu.VMEM((k, n), jnp.bfloat16),
            pltpu.VMEM((m, n), jnp.bfloat16),
        ] + stream_scratch,
        compiler_params=pltpu.CompilerParams(collective_id=0),
    )(A, B, params)


# device time: 65703 ns/iter; 1.1406x vs baseline; 1.1406x over previous
import jax
import jax.numpy as jnp
from jax import lax
from jax.experimental import pallas as pl
from jax.experimental.pallas import tpu as pltpu

N_DEV = 16
S = 2

RING = [0, 1, 5, 9, 13, 14, 10, 6, 2, 3, 7, 11, 15, 12, 8, 4]

CW_STEPS = N_DEV // 2
CCW_STEPS = N_DEV // 2 - 1


def kernel(A, B):
    m, k = A.shape
    _, n = B.shape
    chunk = m // N_DEV
    S_eff = max(1, min(S, n // 256))
    w = n // S_eff

    my = lax.axis_index("i")
    r_arr = jnp.array(RING, jnp.int32)
    kpos = jnp.argmax(r_arr == my).astype(jnp.int32)
    succ = r_arr[(kpos + 1) % N_DEV]
    pred = r_arr[(kpos - 1) % N_DEV]
    offs = jnp.array([8] + [7 - h for h in range(CW_STEPS)]
                     + [-7] + [-6 + h for h in range(CCW_STEPS)], jnp.int32)
    ids = r_arr[(kpos + offs) % N_DEV]
    params = jnp.concatenate([succ[None], pred[None], ids]).astype(jnp.int32)

    def body(a_ref, b_ref, prm_ref, out_ref, ab_ref, bb_ref, pc_cw, pc_ccw,
             *stream_refs):
        succ = prm_ref[0]
        pred = prm_ref[1]

        streams = []
        metas = [(d, j2 * w) for j2 in range(S_eff) for d in (0, 1)]
        for j, (d, lo) in enumerate(metas):
            buf, snd, rcv = stream_refs[3 * j: 3 * j + 3]
            streams.append(dict(
                buf=buf, snd=snd, rcv=rcv, d=d, lo=lo,
                dst=succ if d == 0 else pred,
                steps=CW_STEPS if d == 0 else CCW_STEPS,
                pc=pc_cw if d == 0 else pc_ccw,
                rdmas=[],
            ))

        barrier_sem = pltpu.get_barrier_semaphore()
        for nbr in (succ, pred):
            pl.semaphore_signal(barrier_sem, inc=1, device_id=(nbr,),
                                device_id_type=pl.DeviceIdType.MESH)

        ab_ref[:, :] = a_ref[:, :].astype(jnp.bfloat16)
        bb_ref[:, :] = b_ref[:, :].astype(jnp.bfloat16)

        def chunk_mm(c, out):
            out[:, :] = jnp.dot(
                ab_ref[pl.ds(c * chunk, chunk), :],
                bb_ref[:, :],
                preferred_element_type=jnp.float32,
            ).astype(jnp.bfloat16)

        def start_hop(s, h):
            rd = pltpu.make_async_remote_copy(
                src_ref=s['buf'].at[h],
                dst_ref=s['buf'].at[h + 1],
                send_sem=s['snd'].at[h],
                recv_sem=s['rcv'].at[h],
                device_id=(s['dst'],),
                device_id_type=pl.DeviceIdType.MESH,
            )
            rd.start()
            s['rdmas'].append(rd)

        chunk_mm(prm_ref[2], pc_cw)
        chunk_mm(prm_ref[11], pc_ccw)
        for s in streams:
            s['buf'][0, :, :] = s['pc'][:, s['lo']:s['lo'] + w]
        pl.semaphore_wait(barrier_sem, 2)
        for s in streams:
            start_hop(s, 0)
        chunk_mm(prm_ref[3], pc_cw)
        chunk_mm(prm_ref[12], pc_ccw)

        for h in range(CW_STEPS):
            for s in streams:
                if h >= s['steps']:
                    continue
                s['rdmas'][h].wait_recv()
                if h < s['steps'] - 1:
                    s['buf'][h + 1, :, :] = (
                        s['buf'][h + 1, :, :]
                        + s['pc'][:, s['lo']:s['lo'] + w])
                    start_hop(s, h + 1)
            if h + 1 < CW_STEPS:
                chunk_mm(prm_ref[3 + h + 1], pc_cw)
            if h + 1 < CCW_STEPS - 1:
                chunk_mm(prm_ref[12 + h + 1], pc_ccw)

        for j, (d, lo) in enumerate(metas):
            if d != 0:
                continue
            s_cw = streams[j]
            s_ccw = streams[j + 1]
            assert s_ccw['d'] == 1 and s_ccw['lo'] == lo
            out_ref[:, lo:lo + w] = (
                s_cw['buf'][CW_STEPS, :, :].astype(jnp.float32)
                + s_ccw['buf'][CCW_STEPS, :, :].astype(jnp.float32)
                + pc_cw[:, lo:lo + w].astype(jnp.float32))

        for s in streams:
            for rd in s['rdmas']:
                rd.wait_send()

    stream_scratch = []
    for d_, lo_ in [(d, j2 * w) for j2 in range(S_eff) for d in (0, 1)]:
        steps = CW_STEPS if d_ == 0 else CCW_STEPS
        stream_scratch += [
            pltpu.VMEM((steps + 1, chunk, w), jnp.bfloat16),
            pltpu.SemaphoreType.DMA((steps,)),
            pltpu.SemaphoreType.DMA((steps,)),
        ]

    return pl.pallas_call(
        body,
        out_shape=jax.ShapeDtypeStruct((chunk, n), jnp.float32),
        in_specs=[
            pl.BlockSpec(memory_space=pltpu.VMEM),
            pl.BlockSpec(memory_space=pltpu.VMEM),
            pl.BlockSpec(memory_space=pltpu.SMEM),
        ],
        out_specs=pl.BlockSpec(memory_space=pltpu.VMEM),
        scratch_shapes=[
            pltpu.VMEM((m, k), jnp.bfloat16),
            pltpu.VMEM((k, n), jnp.bfloat16),
            pltpu.VMEM((chunk, n), jnp.bfloat16),
            pltpu.VMEM((chunk, n), jnp.bfloat16),
        ] + stream_scratch,
        compiler_params=pltpu.CompilerParams(collective_id=0),
    )(A, B, params)
